# baseline (speedup 1.0000x reference)
import jax
import jax.numpy as jnp
import numpy as np
from jax.experimental import pallas as pl
from jax.experimental.pallas import tpu as pltpu

K = 20


def _knn_idx(x, k):
    xt = jnp.transpose(x, (0, 2, 1))
    inner = jnp.matmul(xt, x)
    sq = jnp.sum(x * x, axis=1)
    neg_dist = 2.0 * inner - sq[:, :, None] - sq[:, None, :]
    return jax.lax.top_k(neg_dist, k)[1]


def _gather_nb(x, idx):
    xt = jnp.transpose(x, (0, 2, 1))
    nb = jax.vmap(lambda a, i: a[i])(xt, idx)
    return jnp.transpose(nb, (0, 3, 1, 2))


def _bn(x, gamma, beta, axes, pshape):
    mean = jnp.mean(x, axis=axes, keepdims=True)
    var = jnp.var(x, axis=axes, keepdims=True)
    xn = (x - mean) * jax.lax.rsqrt(var + 1e-5)
    return xn * gamma.reshape(pshape) + beta.reshape(pshape)


def _lrelu(x):
    return jnp.where(x >= 0, x, 0.2 * x)


def _conv_bn(feat, p):
    out = jnp.einsum('oc,bcnk->bonk', p['W'], feat)
    return _lrelu(_bn(out, p['gamma'], p['beta'], (0, 2, 3), (1, -1, 1, 1)))


def _gec_layer1(xyz, nrm, p, k):
    idx = _knn_idx(xyz, k)
    xyz_j = _gather_nb(xyz, idx)
    n_j = _gather_nb(nrm, idx)
    xyz_i = jnp.broadcast_to(xyz[:, :, :, None], xyz_j.shape)
    n_i = jnp.broadcast_to(nrm[:, :, :, None], n_j.shape)
    rel = xyz_j - xyz_i
    dist = jnp.sqrt(jnp.sum(rel * rel, axis=1, keepdims=True) + 1e-12)
    dotn = jnp.sum(n_i * n_j, axis=1, keepdims=True)
    feat = jnp.concatenate([xyz_i, rel, dist, n_i, n_j, dotn], axis=1)
    return _conv_bn(feat, p)


def _gec_dyn(x, p, k):
    idx = _knn_idx(x, k)
    x_j = _gather_nb(x, idx)
    x_i = jnp.broadcast_to(x[:, :, :, None], x_j.shape)
    feat = jnp.concatenate([x_i, x_j - x_i], axis=1)
    return _conv_bn(feat, p)


def _head_kernel(h_ref, l1_ref, l2_ref, l2b_ref, l3_ref, l3b_ref, out_ref):
    h = h_ref[...]
    p1 = jnp.max(h, axis=-1)
    p2 = jnp.mean(h, axis=-1)
    z = jnp.concatenate([p1, p2], axis=1)
    z = jnp.dot(z, l1_ref[...].T, preferred_element_type=jnp.float32)
    mean = jnp.mean(z, axis=0, keepdims=True)
    var = jnp.mean((z - mean) * (z - mean), axis=0, keepdims=True)
    z = (z - mean) * jax.lax.rsqrt(var + 1e-5)
    z = jnp.where(z >= 0, z, 0.2 * z)
    z = jnp.dot(z, l2_ref[...].T, preferred_element_type=jnp.float32) + l2b_ref[...]
    mean = jnp.mean(z, axis=0, keepdims=True)
    var = jnp.mean((z - mean) * (z - mean), axis=0, keepdims=True)
    z = (z - mean) * jax.lax.rsqrt(var + 1e-5)
    z = jnp.where(z >= 0, z, 0.2 * z)
    out_ref[...] = jnp.dot(z, l3_ref[...].T, preferred_element_type=jnp.float32) + l3b_ref[...]


def kernel(x, n, params):
    g = params['gec']
    feat = _gec_layer1(x, n, g[0], K)
    resx = feat
    x1 = jnp.max(feat, axis=-1)
    feat = _gec_dyn(x1, g[1], K) + resx
    x2 = jnp.max(feat, axis=-1)
    feat = _gec_dyn(x2, g[2], K)
    resx = feat
    x3 = jnp.max(feat, axis=-1)
    feat = _gec_dyn(x3, g[3], K) + resx
    x4 = jnp.max(feat, axis=-1)
    feat = _gec_dyn(x4, g[4], K)
    resx = feat
    x5 = jnp.max(feat, axis=-1)
    feat = _gec_dyn(x5, g[5], K) + resx
    x6 = jnp.max(feat, axis=-1)
    feat = _gec_dyn(x6, g[6], K)
    resx = feat
    x7 = jnp.max(feat, axis=-1)
    feat = _gec_dyn(x7, g[7], K) + resx
    x8 = jnp.max(feat, axis=-1)
    cat = jnp.concatenate([x1, x2, x3, x4, x5, x6, x7, x8], axis=1)
    h = jnp.einsum('oc,bcn->bon', params['conv4_W'], cat)
    h = _lrelu(_bn(h, params['conv4_gamma'], params['conv4_beta'], (0, 2), (1, -1, 1)))

    out = pl.pallas_call(
        _head_kernel,
        out_shape=jax.ShapeDtypeStruct((h.shape[0], 40), jnp.float32),
    )(h, params['lin1_W'], params['lin2_W'], params['lin2_b'],
      params['lin3_W'], params['lin3_b'])
    return out


# speedup vs baseline: 1.2777x; 1.2777x over previous
import jax
import jax.numpy as jnp
import numpy as np
from jax.experimental import pallas as pl
from jax.experimental.pallas import tpu as pltpu

K = 20
N = 1024


def _knn_body(xt_ref, x_ref, idx_ref, vals_ref, dist_ref):
    xt = xt_ref[0]
    x = x_ref[0]
    inner = jnp.dot(xt, x, preferred_element_type=jnp.float32)
    sq = jnp.sum(x * x, axis=0)
    nd = 2.0 * inner - sq[:, None] - sq[None, :]
    dist_ref[...] = nd
    iota_j = jax.lax.broadcasted_iota(jnp.int32, (N, N), 1)
    idx_cols, val_cols = [], []
    for _ in range(K):
        d = dist_ref[...]
        m = jnp.max(d, axis=1, keepdims=True)
        key = jnp.where(d == m, iota_j, N)
        j = jnp.min(key, axis=1, keepdims=True)
        idx_cols.append(j)
        val_cols.append(m)
        dist_ref[...] = jnp.where(iota_j == j, -jnp.inf, d)
    idx_ref[0] = jnp.concatenate(idx_cols, axis=1)
    vals_ref[0] = jnp.concatenate(val_cols, axis=1)


def _knn_pallas(x):
    B, C, _ = x.shape
    xt = jnp.transpose(x, (0, 2, 1))
    return pl.pallas_call(
        _knn_body,
        grid=(B,),
        in_specs=[
            pl.BlockSpec((1, N, C), lambda b: (b, 0, 0)),
            pl.BlockSpec((1, C, N), lambda b: (b, 0, 0)),
        ],
        out_specs=[
            pl.BlockSpec((1, N, K), lambda b: (b, 0, 0)),
            pl.BlockSpec((1, N, K), lambda b: (b, 0, 0)),
        ],
        out_shape=[
            jax.ShapeDtypeStruct((B, N, K), jnp.int32),
            jax.ShapeDtypeStruct((B, N, K), jnp.float32),
        ],
        scratch_shapes=[pltpu.VMEM((N, N), jnp.float32)],
    )(xt, x)


def _knn_idx(x, k):
    assert k == K
    return _knn_pallas(x)[0]


def _gather_nb(x, idx):
    xt = jnp.transpose(x, (0, 2, 1))
    nb = jax.vmap(lambda a, i: a[i])(xt, idx)
    return jnp.transpose(nb, (0, 3, 1, 2))


def _bn(x, gamma, beta, axes, pshape):
    mean = jnp.mean(x, axis=axes, keepdims=True)
    var = jnp.var(x, axis=axes, keepdims=True)
    xn = (x - mean) * jax.lax.rsqrt(var + 1e-5)
    return xn * gamma.reshape(pshape) + beta.reshape(pshape)


def _lrelu(x):
    return jnp.where(x >= 0, x, 0.2 * x)


def _conv_bn(feat, p):
    out = jnp.einsum('oc,bcnk->bonk', p['W'], feat)
    return _lrelu(_bn(out, p['gamma'], p['beta'], (0, 2, 3), (1, -1, 1, 1)))


def _gec_layer1(xyz, nrm, p, k):
    idx = _knn_idx(xyz, k)
    xyz_j = _gather_nb(xyz, idx)
    n_j = _gather_nb(nrm, idx)
    xyz_i = jnp.broadcast_to(xyz[:, :, :, None], xyz_j.shape)
    n_i = jnp.broadcast_to(nrm[:, :, :, None], n_j.shape)
    rel = xyz_j - xyz_i
    dist = jnp.sqrt(jnp.sum(rel * rel, axis=1, keepdims=True) + 1e-12)
    dotn = jnp.sum(n_i * n_j, axis=1, keepdims=True)
    feat = jnp.concatenate([xyz_i, rel, dist, n_i, n_j, dotn], axis=1)
    return _conv_bn(feat, p)


def _gec_dyn(x, p, k):
    idx = _knn_idx(x, k)
    x_j = _gather_nb(x, idx)
    x_i = jnp.broadcast_to(x[:, :, :, None], x_j.shape)
    feat = jnp.concatenate([x_i, x_j - x_i], axis=1)
    return _conv_bn(feat, p)


def _head_kernel(h_ref, l1_ref, l2_ref, l2b_ref, l3_ref, l3b_ref, out_ref):
    h = h_ref[...]
    p1 = jnp.max(h, axis=-1)
    p2 = jnp.mean(h, axis=-1)
    z = jnp.concatenate([p1, p2], axis=1)
    z = jnp.dot(z, l1_ref[...].T, preferred_element_type=jnp.float32)
    mean = jnp.mean(z, axis=0, keepdims=True)
    var = jnp.mean((z - mean) * (z - mean), axis=0, keepdims=True)
    z = (z - mean) * jax.lax.rsqrt(var + 1e-5)
    z = jnp.where(z >= 0, z, 0.2 * z)
    z = jnp.dot(z, l2_ref[...].T, preferred_element_type=jnp.float32) + l2b_ref[...]
    mean = jnp.mean(z, axis=0, keepdims=True)
    var = jnp.mean((z - mean) * (z - mean), axis=0, keepdims=True)
    z = (z - mean) * jax.lax.rsqrt(var + 1e-5)
    z = jnp.where(z >= 0, z, 0.2 * z)
    out_ref[...] = jnp.dot(z, l3_ref[...].T, preferred_element_type=jnp.float32) + l3b_ref[...]


def kernel(x, n, params):
    g = params['gec']
    feat = _gec_layer1(x, n, g[0], K)
    resx = feat
    x1 = jnp.max(feat, axis=-1)
    feat = _gec_dyn(x1, g[1], K) + resx
    x2 = jnp.max(feat, axis=-1)
    feat = _gec_dyn(x2, g[2], K)
    resx = feat
    x3 = jnp.max(feat, axis=-1)
    feat = _gec_dyn(x3, g[3], K) + resx
    x4 = jnp.max(feat, axis=-1)
    feat = _gec_dyn(x4, g[4], K)
    resx = feat
    x5 = jnp.max(feat, axis=-1)
    feat = _gec_dyn(x5, g[5], K) + resx
    x6 = jnp.max(feat, axis=-1)
    feat = _gec_dyn(x6, g[6], K)
    resx = feat
    x7 = jnp.max(feat, axis=-1)
    feat = _gec_dyn(x7, g[7], K) + resx
    x8 = jnp.max(feat, axis=-1)
    cat = jnp.concatenate([x1, x2, x3, x4, x5, x6, x7, x8], axis=1)
    h = jnp.einsum('oc,bcn->bon', params['conv4_W'], cat)
    h = _lrelu(_bn(h, params['conv4_gamma'], params['conv4_beta'], (0, 2), (1, -1, 1)))

    out = pl.pallas_call(
        _head_kernel,
        out_shape=jax.ShapeDtypeStruct((h.shape[0], 40), jnp.float32),
    )(h, params['lin1_W'], params['lin2_W'], params['lin2_b'],
      params['lin3_W'], params['lin3_b'])
    return out
